# Initial kernel scaffold; baseline (speedup 1.0000x reference)
import jax
import jax.numpy as jnp
from jax.experimental import pallas as pl


def kernel(x):
    raise NotImplementedError("write your pallas kernel here")


# speedup vs baseline: 6.4844x; 6.4844x over previous
import functools
import math

import jax
import jax.numpy as jnp
import numpy as np
from jax.experimental import pallas as pl
from jax.experimental.pallas import tpu as pltpu

_HI = jax.lax.Precision.HIGHEST

_Y_TABLE = np.array(
    [[16, 11, 10, 16, 24, 40, 51, 61],
     [12, 12, 14, 19, 26, 58, 60, 55],
     [14, 13, 16, 24, 40, 57, 69, 56],
     [14, 17, 22, 29, 51, 87, 80, 62],
     [18, 22, 37, 56, 68, 109, 103, 77],
     [24, 35, 55, 64, 81, 104, 113, 92],
     [49, 64, 78, 87, 103, 121, 120, 101],
     [72, 92, 95, 98, 112, 100, 103, 99]], dtype=np.float64).T

_C_TABLE = np.full((8, 8), 99, dtype=np.float64)
_C_TABLE[:4, :4] = np.array(
    [[17, 18, 24, 47], [18, 21, 26, 66],
     [24, 26, 56, 99], [47, 66, 99, 99]], dtype=np.float64).T


def _factor(quality: float) -> float:
    if quality < 50:
        quality = 5000.0 / quality
    else:
        quality = 200.0 - quality * 2
    return quality / 100.0


def _dct8() -> np.ndarray:
    a = np.array([1.0 / math.sqrt(2.0)] + [1.0] * 7, dtype=np.float64)
    f = np.arange(8.0)[:, None]
    p = np.arange(8.0)[None, :]
    return 0.5 * a[:, None] * np.cos((2 * p + 1) * f * math.pi / 16.0)


@functools.cache
def _consts(h: int, w: int, quality: float):
    fac = _factor(quality)
    c8 = _dct8()

    def blockdiag(n):
        return np.kron(np.eye(n // 8), c8)

    def pool(n):
        p = np.zeros((n // 2, n), dtype=np.float64)
        idx = np.arange(n // 2)
        p[idx, 2 * idx] = 0.5
        p[idx, 2 * idx + 1] = 0.5
        return p

    ch = blockdiag(h)
    cw = blockdiag(w)
    eh = blockdiag(h // 2) @ pool(h)
    ew = blockdiag(w // 2) @ pool(w)

    qy = np.tile(_Y_TABLE * fac, (h // 8, w // 8))
    qc = np.tile(_C_TABLE * fac, (h // 16, w // 16))

    arrs = (ch, ch.T, cw, cw.T, eh, eh.T, ew, ew.T,
            qy, 1.0 / qy, qc, 1.0 / qc)
    return tuple(a.astype(np.float32) for a in arrs)


def _diffjpeg_body(x_ref, ch_ref, cht_ref, cw_ref, cwt_ref,
                   eh_ref, eht_ref, ew_ref, ewt_ref,
                   qy_ref, qyi_ref, qc_ref, qci_ref, o_ref):
    def mm(a, b):
        return jnp.dot(a, b, precision=_HI, preferred_element_type=jnp.float32)

    def qround(t, qi_ref, q_ref):
        t = t * qi_ref[...]
        r = jnp.round(t)
        return (r + (t - r) ** 3) * q_ref[...]

    r = x_ref[0, 0] * 255.0
    g = x_ref[0, 1] * 255.0
    b = x_ref[0, 2] * 255.0

    y = 0.299 * r + 0.587 * g + 0.114 * b - 128.0
    cb = -0.168736 * r - 0.331264 * g + 0.5 * b
    cr = 0.5 * r - 0.418688 * g - 0.081312 * b

    ty = qround(mm(mm(ch_ref[...], y), cwt_ref[...]), qyi_ref, qy_ref)
    y_rec = mm(mm(cht_ref[...], ty), cw_ref[...]) + 128.0

    def chroma(c):
        t = qround(mm(mm(eh_ref[...], c), ewt_ref[...]), qci_ref, qc_ref)
        return 4.0 * mm(mm(eht_ref[...], t), ew_ref[...])

    cb_rec = chroma(cb)
    cr_rec = chroma(cr)

    inv255 = jnp.float32(1.0 / 255.0)
    r_out = y_rec + 1.402 * cr_rec
    g_out = y_rec - 0.344136 * cb_rec - 0.714136 * cr_rec
    b_out = y_rec + 1.772 * cb_rec
    o_ref[0, 0] = jnp.clip(r_out, 0.0, 255.0) * inv255
    o_ref[0, 1] = jnp.clip(g_out, 0.0, 255.0) * inv255
    o_ref[0, 2] = jnp.clip(b_out, 0.0, 255.0) * inv255


@jax.jit
def _diffjpeg(x):
    bsz, c, h, w = x.shape
    assert c == 3 and h % 16 == 0 and w % 16 == 0
    consts = [jnp.asarray(a) for a in _consts(h, w, 75.0)]

    img_spec = pl.BlockSpec((1, 3, h, w), lambda i: (i, 0, 0, 0))
    const_specs = [
        pl.BlockSpec(a.shape, lambda i, n=a.ndim: (0,) * n)
        for a in consts
    ]
    return pl.pallas_call(
        _diffjpeg_body,
        out_shape=jax.ShapeDtypeStruct((bsz, 3, h, w), jnp.float32),
        grid=(bsz,),
        in_specs=[img_spec] + const_specs,
        out_specs=img_spec,
        compiler_params=pltpu.CompilerParams(
            dimension_semantics=("parallel",),
            vmem_limit_bytes=96 * 1024 * 1024,
        ),
    )(x, *consts)


def kernel(x):
    return _diffjpeg(x)


# speedup vs baseline: 7.4385x; 1.1471x over previous
import functools
import math

import jax
import jax.numpy as jnp
import numpy as np
from jax.experimental import pallas as pl
from jax.experimental.pallas import tpu as pltpu

_HI = jax.lax.Precision.HIGHEST

_Y_TABLE = np.array(
    [[16, 11, 10, 16, 24, 40, 51, 61],
     [12, 12, 14, 19, 26, 58, 60, 55],
     [14, 13, 16, 24, 40, 57, 69, 56],
     [14, 17, 22, 29, 51, 87, 80, 62],
     [18, 22, 37, 56, 68, 109, 103, 77],
     [24, 35, 55, 64, 81, 104, 113, 92],
     [49, 64, 78, 87, 103, 121, 120, 101],
     [72, 92, 95, 98, 112, 100, 103, 99]], dtype=np.float64).T

_C_TABLE = np.full((8, 8), 99, dtype=np.float64)
_C_TABLE[:4, :4] = np.array(
    [[17, 18, 24, 47], [18, 21, 26, 66],
     [24, 26, 56, 99], [47, 66, 99, 99]], dtype=np.float64).T


def _factor(quality: float) -> float:
    if quality < 50:
        quality = 5000.0 / quality
    else:
        quality = 200.0 - quality * 2
    return quality / 100.0


def _dct8() -> np.ndarray:
    a = np.array([1.0 / math.sqrt(2.0)] + [1.0] * 7, dtype=np.float64)
    f = np.arange(8.0)[:, None]
    p = np.arange(8.0)[None, :]
    return 0.5 * a[:, None] * np.cos((2 * p + 1) * f * math.pi / 16.0)


@functools.cache
def _consts(h: int, w: int, quality: float):
    fac = _factor(quality)
    c8 = _dct8()

    def blockdiag(n):
        return np.kron(np.eye(n // 8), c8)

    def pool(n):
        p = np.zeros((n // 2, n), dtype=np.float64)
        idx = np.arange(n // 2)
        p[idx, 2 * idx] = 0.5
        p[idx, 2 * idx + 1] = 0.5
        return p

    ch = blockdiag(h)
    cw = blockdiag(w)
    eh = blockdiag(h // 2) @ pool(h)
    ew = blockdiag(w // 2) @ pool(w)

    qy = np.tile(_Y_TABLE * fac, (h // 8, w // 8))
    qc = np.tile(_C_TABLE * fac, (h // 16, w // 16))

    arrs = (ch, ch.T, cw, cw.T, eh, eh.T, ew, ew.T,
            qy, 1.0 / qy, qc, 1.0 / qc)
    return tuple(a.astype(np.float32) for a in arrs)


def _diffjpeg_body(x_ref, ch_ref, cht_ref, cw_ref, cwt_ref,
                   eh_ref, eht_ref, ew_ref, ewt_ref,
                   qy_ref, qyi_ref, qc_ref, qci_ref, o_ref):
    def mm(a, b):
        return jnp.dot(a, b, precision=_HI, preferred_element_type=jnp.float32)

    def qround(t, qi_ref, q_ref):
        t = t * qi_ref[...]
        r = jnp.round(t)
        return (r + (t - r) ** 3) * q_ref[...]

    for i in range(x_ref.shape[0]):
        r = x_ref[i, 0] * 255.0
        g = x_ref[i, 1] * 255.0
        b = x_ref[i, 2] * 255.0

        y = 0.299 * r + 0.587 * g + 0.114 * b - 128.0
        cb = -0.168736 * r - 0.331264 * g + 0.5 * b
        cr = 0.5 * r - 0.418688 * g - 0.081312 * b

        ty = qround(mm(mm(ch_ref[...], y), cwt_ref[...]), qyi_ref, qy_ref)
        y_rec = mm(mm(cht_ref[...], ty), cw_ref[...]) + 128.0

        def chroma(c):
            t = qround(mm(mm(eh_ref[...], c), ewt_ref[...]), qci_ref, qc_ref)
            return 4.0 * mm(mm(eht_ref[...], t), ew_ref[...])

        cb_rec = chroma(cb)
        cr_rec = chroma(cr)

        inv255 = jnp.float32(1.0 / 255.0)
        r_out = y_rec + 1.402 * cr_rec
        g_out = y_rec - 0.344136 * cb_rec - 0.714136 * cr_rec
        b_out = y_rec + 1.772 * cb_rec
        o_ref[i, 0] = jnp.clip(r_out, 0.0, 255.0) * inv255
        o_ref[i, 1] = jnp.clip(g_out, 0.0, 255.0) * inv255
        o_ref[i, 2] = jnp.clip(b_out, 0.0, 255.0) * inv255


@jax.jit
def _diffjpeg(x):
    bsz, c, h, w = x.shape
    assert c == 3 and h % 16 == 0 and w % 16 == 0
    consts = [jnp.asarray(a) for a in _consts(h, w, 75.0)]

    gsz = 4 if bsz % 4 == 0 else 1
    img_spec = pl.BlockSpec((gsz, 3, h, w), lambda i: (i, 0, 0, 0))
    const_specs = [
        pl.BlockSpec(a.shape, lambda i, n=a.ndim: (0,) * n)
        for a in consts
    ]
    return pl.pallas_call(
        _diffjpeg_body,
        out_shape=jax.ShapeDtypeStruct((bsz, 3, h, w), jnp.float32),
        grid=(bsz // gsz,),
        in_specs=[img_spec] + const_specs,
        out_specs=img_spec,
        compiler_params=pltpu.CompilerParams(
            dimension_semantics=("parallel",),
            vmem_limit_bytes=60 * 1024 * 1024,
        ),
    )(x, *consts)


def kernel(x):
    return _diffjpeg(x)


# speedup vs baseline: 7.5135x; 1.0101x over previous
import functools
import math

import jax
import jax.numpy as jnp
import numpy as np
from jax.experimental import pallas as pl
from jax.experimental.pallas import tpu as pltpu

_HI = jax.lax.Precision.HIGHEST

_Y_TABLE = np.array(
    [[16, 11, 10, 16, 24, 40, 51, 61],
     [12, 12, 14, 19, 26, 58, 60, 55],
     [14, 13, 16, 24, 40, 57, 69, 56],
     [14, 17, 22, 29, 51, 87, 80, 62],
     [18, 22, 37, 56, 68, 109, 103, 77],
     [24, 35, 55, 64, 81, 104, 113, 92],
     [49, 64, 78, 87, 103, 121, 120, 101],
     [72, 92, 95, 98, 112, 100, 103, 99]], dtype=np.float64).T

_C_TABLE = np.full((8, 8), 99, dtype=np.float64)
_C_TABLE[:4, :4] = np.array(
    [[17, 18, 24, 47], [18, 21, 26, 66],
     [24, 26, 56, 99], [47, 66, 99, 99]], dtype=np.float64).T


def _factor(quality: float) -> float:
    if quality < 50:
        quality = 5000.0 / quality
    else:
        quality = 200.0 - quality * 2
    return quality / 100.0


def _dct8() -> np.ndarray:
    a = np.array([1.0 / math.sqrt(2.0)] + [1.0] * 7, dtype=np.float64)
    f = np.arange(8.0)[:, None]
    p = np.arange(8.0)[None, :]
    return 0.5 * a[:, None] * np.cos((2 * p + 1) * f * math.pi / 16.0)


@functools.cache
def _consts(h: int, w: int, quality: float):
    fac = _factor(quality)
    c8 = _dct8()

    def blockdiag(n):
        return np.kron(np.eye(n // 8), c8)

    def pool(n):
        p = np.zeros((n // 2, n), dtype=np.float64)
        idx = np.arange(n // 2)
        p[idx, 2 * idx] = 0.5
        p[idx, 2 * idx + 1] = 0.5
        return p

    ch = blockdiag(h)
    cw = blockdiag(w)
    eh = blockdiag(h // 2) @ pool(h)
    ew = blockdiag(w // 2) @ pool(w)

    qy = np.tile(_Y_TABLE * fac, (h // 8, w // 8))
    qc = np.tile(_C_TABLE * fac, (h // 16, w // 16))

    arrs = (ch, ch.T, cw, cw.T, eh, eh.T, ew, ew.T,
            qy, 1.0 / qy, qc, 1.0 / qc)
    return tuple(a.astype(np.float32) for a in arrs)


def _diffjpeg_body(x_ref, ch_ref, cht_ref, cw_ref, cwt_ref,
                   eh_ref, eht_ref, ew_ref, ewt_ref,
                   qy_ref, qyi_ref, qc_ref, qci_ref, o_ref):
    def mm(a, b):
        return jnp.dot(a, b, precision=_HI, preferred_element_type=jnp.float32)

    def qround(t, qi_ref, q_ref):
        t = t * qi_ref[...]
        r = jnp.round(t)
        return (r + (t - r) ** 3) * q_ref[...]

    for i in range(x_ref.shape[0]):
        r = x_ref[i, 0] * 255.0
        g = x_ref[i, 1] * 255.0
        b = x_ref[i, 2] * 255.0

        y = 0.299 * r + 0.587 * g + 0.114 * b - 128.0
        cb = -0.168736 * r - 0.331264 * g + 0.5 * b
        cr = 0.5 * r - 0.418688 * g - 0.081312 * b

        ty = qround(mm(mm(ch_ref[...], y), cwt_ref[...]), qyi_ref, qy_ref)
        y_rec = mm(mm(cht_ref[...], ty), cw_ref[...]) + 128.0

        def chroma(c):
            t = qround(mm(mm(eh_ref[...], c), ewt_ref[...]), qci_ref, qc_ref)
            return 4.0 * mm(mm(eht_ref[...], t), ew_ref[...])

        cb_rec = chroma(cb)
        cr_rec = chroma(cr)

        inv255 = jnp.float32(1.0 / 255.0)
        r_out = y_rec + 1.402 * cr_rec
        g_out = y_rec - 0.344136 * cb_rec - 0.714136 * cr_rec
        b_out = y_rec + 1.772 * cb_rec
        o_ref[i, 0] = jnp.clip(r_out, 0.0, 255.0) * inv255
        o_ref[i, 1] = jnp.clip(g_out, 0.0, 255.0) * inv255
        o_ref[i, 2] = jnp.clip(b_out, 0.0, 255.0) * inv255


@jax.jit
def _diffjpeg(x):
    bsz, c, h, w = x.shape
    assert c == 3 and h % 16 == 0 and w % 16 == 0
    consts = [jnp.asarray(a) for a in _consts(h, w, 75.0)]

    gsz = 8 if bsz % 8 == 0 else 1
    img_spec = pl.BlockSpec((gsz, 3, h, w), lambda i: (i, 0, 0, 0))
    const_specs = [
        pl.BlockSpec(a.shape, lambda i, n=a.ndim: (0,) * n)
        for a in consts
    ]
    return pl.pallas_call(
        _diffjpeg_body,
        out_shape=jax.ShapeDtypeStruct((bsz, 3, h, w), jnp.float32),
        grid=(bsz // gsz,),
        in_specs=[img_spec] + const_specs,
        out_specs=img_spec,
        compiler_params=pltpu.CompilerParams(
            dimension_semantics=("parallel",),
            vmem_limit_bytes=60 * 1024 * 1024,
        ),
    )(x, *consts)


def kernel(x):
    return _diffjpeg(x)


# speedup vs baseline: 9.4307x; 1.2552x over previous
import functools
import math

import jax
import jax.numpy as jnp
import numpy as np
from jax.experimental import pallas as pl
from jax.experimental.pallas import tpu as pltpu

_HI = jax.lax.Precision.HIGHEST

_Y_TABLE = np.array(
    [[16, 11, 10, 16, 24, 40, 51, 61],
     [12, 12, 14, 19, 26, 58, 60, 55],
     [14, 13, 16, 24, 40, 57, 69, 56],
     [14, 17, 22, 29, 51, 87, 80, 62],
     [18, 22, 37, 56, 68, 109, 103, 77],
     [24, 35, 55, 64, 81, 104, 113, 92],
     [49, 64, 78, 87, 103, 121, 120, 101],
     [72, 92, 95, 98, 112, 100, 103, 99]], dtype=np.float64).T

_C_TABLE = np.full((8, 8), 99, dtype=np.float64)
_C_TABLE[:4, :4] = np.array(
    [[17, 18, 24, 47], [18, 21, 26, 66],
     [24, 26, 56, 99], [47, 66, 99, 99]], dtype=np.float64).T


def _factor(quality: float) -> float:
    if quality < 50:
        quality = 5000.0 / quality
    else:
        quality = 200.0 - quality * 2
    return quality / 100.0


def _dct8() -> np.ndarray:
    a = np.array([1.0 / math.sqrt(2.0)] + [1.0] * 7, dtype=np.float64)
    f = np.arange(8.0)[:, None]
    p = np.arange(8.0)[None, :]
    return 0.5 * a[:, None] * np.cos((2 * p + 1) * f * math.pi / 16.0)


@functools.cache
def _consts(h: int, w: int, quality: float):
    fac = _factor(quality)
    c8 = _dct8()

    def blockdiag(n):
        return np.kron(np.eye(n // 8), c8)

    def pool(n):
        p = np.zeros((n // 2, n), dtype=np.float64)
        idx = np.arange(n // 2)
        p[idx, 2 * idx] = 0.5
        p[idx, 2 * idx + 1] = 0.5
        return p

    ch = blockdiag(h)
    cw = blockdiag(w)
    eh = blockdiag(h // 2) @ pool(h)
    ew = blockdiag(w // 2) @ pool(w)

    qy = np.tile(_Y_TABLE * fac, (h // 8, w // 8))
    qc = np.tile(_C_TABLE * fac, (h // 16, w // 16))

    f32 = tuple(a.astype(np.float32)
                for a in (ch, cw.T, eh, ew.T, qy, 1.0 / qy, qc, 1.0 / qc))

    def split(a):
        a32 = a.astype(np.float32)
        hi = a32.astype(jnp.bfloat16)
        lo = (a32 - np.asarray(hi, np.float32)).astype(jnp.bfloat16)
        return hi, lo

    bf = split(ch.T) + split(cw) + split(eh.T) + split(ew)
    return f32 + bf


def _diffjpeg_body(x_ref, ch_ref, cwt_ref, eh_ref, ewt_ref,
                   qy_ref, qyi_ref, qc_ref, qci_ref,
                   chth_ref, chtl_ref, cwh_ref, cwl_ref,
                   ehth_ref, ehtl_ref, ewh_ref, ewl_ref, o_ref):
    def mm(a, b):
        return jnp.dot(a, b, precision=_HI, preferred_element_type=jnp.float32)

    def mmbf(a, b):
        return jnp.dot(a, b, preferred_element_type=jnp.float32)

    def mm3(x, wh_ref, wl_ref):
        xh = x.astype(jnp.bfloat16)
        xl = (x - xh.astype(jnp.float32)).astype(jnp.bfloat16)
        wh = wh_ref[...]
        return mmbf(xh, wh) + (mmbf(xh, wl_ref[...]) + mmbf(xl, wh))

    def mm3l(ah_ref, al_ref, x):
        xh = x.astype(jnp.bfloat16)
        xl = (x - xh.astype(jnp.float32)).astype(jnp.bfloat16)
        ah = ah_ref[...]
        return mmbf(ah, xh) + (mmbf(al_ref[...], xh) + mmbf(ah, xl))

    def qround(t, qi_ref, q_ref):
        t = t * qi_ref[...]
        r = jnp.round(t)
        return (r + (t - r) ** 3) * q_ref[...]

    for i in range(x_ref.shape[0]):
        r = x_ref[i, 0] * 255.0
        g = x_ref[i, 1] * 255.0
        b = x_ref[i, 2] * 255.0

        y = 0.299 * r + 0.587 * g + 0.114 * b - 128.0
        cb = -0.168736 * r - 0.331264 * g + 0.5 * b
        cr = 0.5 * r - 0.418688 * g - 0.081312 * b

        ty = qround(mm(mm(ch_ref[...], y), cwt_ref[...]), qyi_ref, qy_ref)
        y_rec = mm3(mm3l(chth_ref, chtl_ref, ty), cwh_ref, cwl_ref) + 128.0

        def chroma(c):
            t = qround(mm(mm(eh_ref[...], c), ewt_ref[...]), qci_ref, qc_ref)
            return 4.0 * mm3(mm3l(ehth_ref, ehtl_ref, t), ewh_ref, ewl_ref)

        cb_rec = chroma(cb)
        cr_rec = chroma(cr)

        inv255 = jnp.float32(1.0 / 255.0)
        r_out = y_rec + 1.402 * cr_rec
        g_out = y_rec - 0.344136 * cb_rec - 0.714136 * cr_rec
        b_out = y_rec + 1.772 * cb_rec
        o_ref[i, 0] = jnp.clip(r_out, 0.0, 255.0) * inv255
        o_ref[i, 1] = jnp.clip(g_out, 0.0, 255.0) * inv255
        o_ref[i, 2] = jnp.clip(b_out, 0.0, 255.0) * inv255


@jax.jit
def _diffjpeg(x):
    bsz, c, h, w = x.shape
    assert c == 3 and h % 16 == 0 and w % 16 == 0
    consts = [jnp.asarray(a) for a in _consts(h, w, 75.0)]

    gsz = 8 if bsz % 8 == 0 else 1
    img_spec = pl.BlockSpec((gsz, 3, h, w), lambda i: (i, 0, 0, 0))
    const_specs = [
        pl.BlockSpec(a.shape, lambda i, n=a.ndim: (0,) * n)
        for a in consts
    ]
    return pl.pallas_call(
        _diffjpeg_body,
        out_shape=jax.ShapeDtypeStruct((bsz, 3, h, w), jnp.float32),
        grid=(bsz // gsz,),
        in_specs=[img_spec] + const_specs,
        out_specs=img_spec,
        compiler_params=pltpu.CompilerParams(
            dimension_semantics=("parallel",),
            vmem_limit_bytes=60 * 1024 * 1024,
        ),
    )(x, *consts)


def kernel(x):
    return _diffjpeg(x)
